# baseline (speedup 1.0000x reference)
import functools

import jax
import jax.numpy as jnp
from jax import lax
from jax.experimental import pallas as pl
from jax.experimental.pallas import tpu as pltpu
from jax.experimental.pallas import tpu_sc as plsc

E = 8
H = 1024
I = 2048
BT = 512
IT = 512
NW = 32
CHUNK = 32


def _router_body(x_ref, gw_ref, logits_ref, idx_ref, p_ref):
    x = x_ref[...]
    gw = gw_ref[...]
    logits = lax.dot_general(x, gw, (((1,), (1,)), ((), ())),
                             preferred_element_type=jnp.float32)
    logits_ref[...] = logits
    m = jnp.max(logits, axis=-1, keepdims=True)
    ex = jnp.exp(logits - m)
    p = ex / jnp.sum(ex, axis=-1, keepdims=True)
    lane = lax.broadcasted_iota(jnp.int32, p.shape, 1)
    m1 = jnp.max(p, axis=-1, keepdims=True)
    i1 = jnp.min(jnp.where(p == m1, lane, E), axis=-1, keepdims=True)
    p2m = jnp.where(lane == i1, -1.0, p)
    m2 = jnp.max(p2m, axis=-1, keepdims=True)
    i2 = jnp.min(jnp.where(p2m == m2, lane, E), axis=-1, keepdims=True)
    idx_ref[...] = jnp.concatenate([i1, i2], axis=1)
    p_ref[...] = jnp.concatenate([m1, m2], axis=1)


def _gmm_body(be_ref, xs_ref, gw_ref, uw_ref, dw_ref, rw_ref, ys_ref,
              acc_ref):
    i = pl.program_id(1)

    @pl.when(i == 0)
    def _():
        acc_ref[...] = jnp.zeros_like(acc_ref)

    xb = xs_ref[...].astype(jnp.bfloat16)
    g = lax.dot_general(xb, gw_ref[0], (((1,), (1,)), ((), ())),
                        preferred_element_type=jnp.float32)
    u = lax.dot_general(xb, uw_ref[0], (((1,), (1,)), ((), ())),
                        preferred_element_type=jnp.float32)
    h = ((g * jax.nn.sigmoid(g)) * u).astype(jnp.bfloat16)
    acc_ref[...] += lax.dot_general(h, dw_ref[0], (((1,), (1,)), ((), ())),
                                    preferred_element_type=jnp.float32)

    @pl.when(i == pl.num_programs(1) - 1)
    def _():
        ys_ref[...] = acc_ref[...] * rw_ref[...]


def _pairsum_body(g_ref, out_ref):
    g = g_ref[...]
    out_ref[...] = g[:, 0, :] + g[:, 1, :]


def _make_sc_gather(m_rows, n_rows):
    nc = n_rows // (NW * CHUNK)
    mesh = plsc.VectorSubcoreMesh(core_axis_name="c", subcore_axis_name="s")

    @functools.partial(
        pl.kernel, mesh=mesh,
        out_type=jax.ShapeDtypeStruct((n_rows, H), jnp.float32),
        scratch_types=[
            pltpu.VMEM((nc, CHUNK), jnp.int32),
            pltpu.VMEM((CHUNK, H), jnp.float32),
            pltpu.VMEM((CHUNK, H), jnp.float32),
            pltpu.SemaphoreType.DMA,
            pltpu.SemaphoreType.DMA,
            pltpu.SemaphoreType.DMA,
            pltpu.SemaphoreType.DMA,
        ],
    )
    def k(src_hbm, idx_hbm, out_hbm, idx_v, rows_v0, rows_v1, gs0, gs1,
          ss0, ss1):
        wid = lax.axis_index("s") * 2 + lax.axis_index("c")
        pltpu.sync_copy(idx_hbm.at[wid], idx_v)
        bufs = (rows_v0, rows_v1)
        gsems = (gs0, gs1)
        ssems = (ss0, ss1)
        gathers = [None] * nc
        stores = [None] * nc
        gathers[0] = pltpu.async_copy(src_hbm.at[idx_v.at[0]], bufs[0],
                                      gsems[0])
        for c in range(nc):
            gathers[c].wait()
            stores[c] = pltpu.async_copy(
                bufs[c % 2],
                out_hbm.at[pl.ds((wid * nc + c) * CHUNK, CHUNK)],
                ssems[c % 2])
            if c + 1 < nc:
                if c >= 1:
                    stores[c - 1].wait()
                gathers[c + 1] = pltpu.async_copy(
                    src_hbm.at[idx_v.at[c + 1]], bufs[(c + 1) % 2],
                    gsems[(c + 1) % 2])
        if nc >= 2:
            stores[nc - 2].wait()
        stores[nc - 1].wait()

    return k


@jax.jit
def _run(x, gate_w, gate_proj_w, up_proj_w, down_proj_w):
    T = x.shape[0]
    A = 2 * T
    A_BUF = A + E * BT
    G = A_BUF // BT
    NT = T // 512
    NI = I // IT

    logits, idx2, p2 = pl.pallas_call(
        _router_body,
        grid=(NT,),
        in_specs=[
            pl.BlockSpec((512, H), lambda t: (t, 0)),
            pl.BlockSpec((E, H), lambda t: (0, 0)),
        ],
        out_specs=[
            pl.BlockSpec((512, E), lambda t: (t, 0)),
            pl.BlockSpec((512, 2), lambda t: (t, 0)),
            pl.BlockSpec((512, 2), lambda t: (t, 0)),
        ],
        out_shape=[
            jax.ShapeDtypeStruct((T, E), jnp.float32),
            jax.ShapeDtypeStruct((T, 2), jnp.int32),
            jax.ShapeDtypeStruct((T, 2), jnp.float32),
        ],
    )(x, gate_w)

    a = idx2.reshape(-1)
    oh = (a[:, None] == jnp.arange(E, dtype=jnp.int32)).astype(jnp.int32)
    cum = jnp.cumsum(oh, axis=0)
    rank = jnp.take_along_axis(cum, a[:, None], axis=1)[:, 0] - 1
    counts = cum[-1]
    nb = (counts + BT - 1) // BT
    cnb = jnp.cumsum(nb)
    blk_start = cnb - nb
    pos = (blk_start[a] * BT + rank).astype(jnp.int32)
    arange_a = jnp.arange(A, dtype=jnp.int32)
    row_token = jnp.zeros((A_BUF,), jnp.int32).at[pos].set(arange_a // 2)
    row_weight = jnp.zeros((A_BUF, 1), jnp.float32).at[pos, 0].set(
        p2.reshape(-1))
    block_expert = jnp.minimum(
        jnp.searchsorted(cnb, jnp.arange(G, dtype=jnp.int32), side="right"),
        E - 1).astype(jnp.int32)

    xs = _make_sc_gather(T, A_BUF)(
        x, row_token.reshape(NW, -1, CHUNK))

    ys = pl.pallas_call(
        _gmm_body,
        grid_spec=pltpu.PrefetchScalarGridSpec(
            num_scalar_prefetch=1,
            grid=(G, NI),
            in_specs=[
                pl.BlockSpec((BT, H), lambda g, i, be: (g, 0)),
                pl.BlockSpec((1, IT, H), lambda g, i, be: (be[g], i, 0)),
                pl.BlockSpec((1, IT, H), lambda g, i, be: (be[g], i, 0)),
                pl.BlockSpec((1, H, IT), lambda g, i, be: (be[g], 0, i)),
                pl.BlockSpec((BT, 1), lambda g, i, be: (g, 0)),
            ],
            out_specs=pl.BlockSpec((BT, H), lambda g, i, be: (g, 0)),
            scratch_shapes=[pltpu.VMEM((BT, H), jnp.float32)],
        ),
        out_shape=jax.ShapeDtypeStruct((A_BUF, H), jnp.float32),
        compiler_params=pltpu.CompilerParams(
            dimension_semantics=("arbitrary", "arbitrary")),
    )(block_expert, xs, gate_proj_w.astype(jnp.bfloat16),
      up_proj_w.astype(jnp.bfloat16), down_proj_w.astype(jnp.bfloat16),
      row_weight)

    g3 = _make_sc_gather(A_BUF, A)(
        ys, pos.reshape(NW, -1, CHUNK)).reshape(T, 2, H)

    final = pl.pallas_call(
        _pairsum_body,
        grid=(NT,),
        in_specs=[pl.BlockSpec((512, 2, H), lambda t: (t, 0, 0))],
        out_specs=pl.BlockSpec((512, H), lambda t: (t, 0)),
        out_shape=jax.ShapeDtypeStruct((T, H), jnp.float32),
    )(g3)

    return final, logits


def kernel(hidden_state, gate_w, gate_proj_w, up_proj_w, down_proj_w):
    b, s, h = hidden_state.shape
    x = hidden_state.reshape(-1, h)
    final, logits = _run(x, gate_w, gate_proj_w, up_proj_w, down_proj_w)
    return final.reshape(b, s, h), logits


# speedup vs baseline: 1.3815x; 1.3815x over previous
import functools

import jax
import jax.numpy as jnp
from jax import lax
from jax.experimental import pallas as pl
from jax.experimental.pallas import tpu as pltpu
from jax.experimental.pallas import tpu_sc as plsc

E = 8
H = 1024
I = 2048
BT = 512
IT = 512
NW = 32
CHUNK = 32


def _router_body(x_ref, gw_ref, logits_ref, idx_ref, p_ref):
    x = x_ref[...]
    gw = gw_ref[...]
    logits = lax.dot_general(x, gw, (((1,), (1,)), ((), ())),
                             preferred_element_type=jnp.float32)
    logits_ref[...] = logits
    m = jnp.max(logits, axis=-1, keepdims=True)
    ex = jnp.exp(logits - m)
    p = ex / jnp.sum(ex, axis=-1, keepdims=True)
    lane = lax.broadcasted_iota(jnp.int32, p.shape, 1)
    m1 = jnp.max(p, axis=-1, keepdims=True)
    i1 = jnp.min(jnp.where(p == m1, lane, E), axis=-1, keepdims=True)
    p2m = jnp.where(lane == i1, -1.0, p)
    m2 = jnp.max(p2m, axis=-1, keepdims=True)
    i2 = jnp.min(jnp.where(p2m == m2, lane, E), axis=-1, keepdims=True)
    idx_ref[...] = jnp.concatenate([i1, i2], axis=1)
    p_ref[...] = jnp.concatenate([m1, m2], axis=1)


def _gmm_body(be_ref, xs_ref, gw_ref, uw_ref, dw_ref, rw_ref, ys_ref,
              acc_ref):
    i = pl.program_id(1)

    @pl.when(i == 0)
    def _():
        acc_ref[...] = jnp.zeros_like(acc_ref)

    xb = xs_ref[...].astype(jnp.bfloat16)
    g = lax.dot_general(xb, gw_ref[0], (((1,), (1,)), ((), ())),
                        preferred_element_type=jnp.float32)
    u = lax.dot_general(xb, uw_ref[0], (((1,), (1,)), ((), ())),
                        preferred_element_type=jnp.float32)
    h = ((g * jax.nn.sigmoid(g)) * u).astype(jnp.bfloat16)
    acc_ref[...] += lax.dot_general(h, dw_ref[0], (((1,), (1,)), ((), ())),
                                    preferred_element_type=jnp.float32)

    @pl.when(i == pl.num_programs(1) - 1)
    def _():
        ys_ref[...] = acc_ref[...] * rw_ref[...]


def _pairsum_body(g_ref, out_ref):
    g = g_ref[...]
    out_ref[...] = g[:, 0, :] + g[:, 1, :]


def _make_sc_gather(m_rows, n_rows):
    nc = n_rows // (NW * CHUNK)
    mesh = plsc.VectorSubcoreMesh(core_axis_name="c", subcore_axis_name="s")

    @functools.partial(
        pl.kernel, mesh=mesh,
        out_type=jax.ShapeDtypeStruct((n_rows, H), jnp.float32),
        scratch_types=[
            pltpu.VMEM((nc, CHUNK), jnp.int32),
            pltpu.VMEM((CHUNK, H), jnp.float32),
            pltpu.VMEM((CHUNK, H), jnp.float32),
            pltpu.SemaphoreType.DMA,
            pltpu.SemaphoreType.DMA,
            pltpu.SemaphoreType.DMA,
            pltpu.SemaphoreType.DMA,
        ],
    )
    def k(src_hbm, idx_hbm, out_hbm, idx_v, rows_v0, rows_v1, gs0, gs1,
          ss0, ss1):
        wid = lax.axis_index("s") * 2 + lax.axis_index("c")
        pltpu.sync_copy(idx_hbm.at[wid], idx_v)
        bufs = (rows_v0, rows_v1)
        gsems = (gs0, gs1)
        ssems = (ss0, ss1)
        gathers = [None] * nc
        stores = [None] * nc
        gathers[0] = pltpu.async_copy(src_hbm.at[idx_v.at[0]], bufs[0],
                                      gsems[0])
        for c in range(nc):
            gathers[c].wait()
            stores[c] = pltpu.async_copy(
                bufs[c % 2],
                out_hbm.at[pl.ds((wid * nc + c) * CHUNK, CHUNK)],
                ssems[c % 2])
            if c + 1 < nc:
                if c >= 1:
                    stores[c - 1].wait()
                gathers[c + 1] = pltpu.async_copy(
                    src_hbm.at[idx_v.at[c + 1]], bufs[(c + 1) % 2],
                    gsems[(c + 1) % 2])
        if nc >= 2:
            stores[nc - 2].wait()
        stores[nc - 1].wait()

    return k


@jax.jit
def _run(x, gate_w, gate_proj_w, up_proj_w, down_proj_w):
    T = x.shape[0]
    A = 2 * T
    A_BUF = A + E * BT
    G = A_BUF // BT
    NT = T // 512
    NI = I // IT

    logits, idx2, p2 = pl.pallas_call(
        _router_body,
        grid=(NT,),
        in_specs=[
            pl.BlockSpec((512, H), lambda t: (t, 0)),
            pl.BlockSpec((E, H), lambda t: (0, 0)),
        ],
        out_specs=[
            pl.BlockSpec((512, E), lambda t: (t, 0)),
            pl.BlockSpec((512, 2), lambda t: (t, 0)),
            pl.BlockSpec((512, 2), lambda t: (t, 0)),
        ],
        out_shape=[
            jax.ShapeDtypeStruct((T, E), jnp.float32),
            jax.ShapeDtypeStruct((T, 2), jnp.int32),
            jax.ShapeDtypeStruct((T, 2), jnp.float32),
        ],
    )(x, gate_w)

    a = idx2.reshape(-1)
    oh = (a[:, None] == jnp.arange(E, dtype=jnp.int32)).astype(jnp.int32)
    cum = jnp.cumsum(oh, axis=0)
    rank = jnp.take_along_axis(cum, a[:, None], axis=1)[:, 0] - 1
    counts = cum[-1]
    nb = (counts + BT - 1) // BT
    cnb = jnp.cumsum(nb)
    blk_start = cnb - nb
    pos = (blk_start[a] * BT + rank).astype(jnp.int32)
    arange_a = jnp.arange(A, dtype=jnp.int32)
    row_token = (jnp.arange(A_BUF, dtype=jnp.int32) % T).at[pos].set(
        arange_a // 2)
    row_weight = jnp.zeros((A_BUF, 1), jnp.float32).at[pos, 0].set(
        p2.reshape(-1))
    block_expert = jnp.minimum(
        jnp.searchsorted(cnb, jnp.arange(G, dtype=jnp.int32), side="right"),
        E - 1).astype(jnp.int32)

    xs = _make_sc_gather(T, A_BUF)(
        x, row_token.reshape(NW, -1, CHUNK))

    ys = pl.pallas_call(
        _gmm_body,
        grid_spec=pltpu.PrefetchScalarGridSpec(
            num_scalar_prefetch=1,
            grid=(G, NI),
            in_specs=[
                pl.BlockSpec((BT, H), lambda g, i, be: (g, 0)),
                pl.BlockSpec((1, IT, H), lambda g, i, be: (be[g], i, 0)),
                pl.BlockSpec((1, IT, H), lambda g, i, be: (be[g], i, 0)),
                pl.BlockSpec((1, H, IT), lambda g, i, be: (be[g], 0, i)),
                pl.BlockSpec((BT, 1), lambda g, i, be: (g, 0)),
            ],
            out_specs=pl.BlockSpec((BT, H), lambda g, i, be: (g, 0)),
            scratch_shapes=[pltpu.VMEM((BT, H), jnp.float32)],
        ),
        out_shape=jax.ShapeDtypeStruct((A_BUF, H), jnp.float32),
        compiler_params=pltpu.CompilerParams(
            dimension_semantics=("arbitrary", "arbitrary")),
    )(block_expert, xs, gate_proj_w.astype(jnp.bfloat16),
      up_proj_w.astype(jnp.bfloat16), down_proj_w.astype(jnp.bfloat16),
      row_weight)

    g3 = _make_sc_gather(A_BUF, A)(
        ys, pos.reshape(NW, -1, CHUNK)).reshape(T, 2, H)

    final = pl.pallas_call(
        _pairsum_body,
        grid=(NT,),
        in_specs=[pl.BlockSpec((512, 2, H), lambda t: (t, 0, 0))],
        out_specs=pl.BlockSpec((512, H), lambda t: (t, 0)),
        out_shape=jax.ShapeDtypeStruct((T, H), jnp.float32),
    )(g3)

    return final, logits


def kernel(hidden_state, gate_w, gate_proj_w, up_proj_w, down_proj_w):
    b, s, h = hidden_state.shape
    x = hidden_state.reshape(-1, h)
    final, logits = _run(x, gate_w, gate_proj_w, up_proj_w, down_proj_w)
    return final.reshape(b, s, h), logits


# speedup vs baseline: 1.5667x; 1.1341x over previous
import functools

import jax
import jax.numpy as jnp
from jax import lax
from jax.experimental import pallas as pl
from jax.experimental.pallas import tpu as pltpu
from jax.experimental.pallas import tpu_sc as plsc

E = 8
H = 1024
I = 2048
BT = 512
IT = 512
NW = 32
CHUNK = 32


def _router_body(x_ref, gw_ref, logits_ref, idx_ref, p_ref):
    x = x_ref[...]
    gw = gw_ref[...]
    logits = lax.dot_general(x, gw, (((1,), (1,)), ((), ())),
                             preferred_element_type=jnp.float32)
    logits_ref[...] = logits
    m = jnp.max(logits, axis=-1, keepdims=True)
    ex = jnp.exp(logits - m)
    p = ex / jnp.sum(ex, axis=-1, keepdims=True)
    lane = lax.broadcasted_iota(jnp.int32, p.shape, 1)
    m1 = jnp.max(p, axis=-1, keepdims=True)
    i1 = jnp.min(jnp.where(p == m1, lane, E), axis=-1, keepdims=True)
    p2m = jnp.where(lane == i1, -1.0, p)
    m2 = jnp.max(p2m, axis=-1, keepdims=True)
    i2 = jnp.min(jnp.where(p2m == m2, lane, E), axis=-1, keepdims=True)
    idx_ref[...] = jnp.concatenate([i1, i2], axis=1)
    p_ref[...] = jnp.concatenate([m1, m2], axis=1)


def _gmm_body(be_ref, xs_ref, gw_ref, uw_ref, dw_ref, ys_ref, acc_ref):
    i = pl.program_id(1)

    @pl.when(i == 0)
    def _():
        acc_ref[...] = jnp.zeros_like(acc_ref)

    xb = xs_ref[...].astype(jnp.bfloat16)
    g = lax.dot_general(xb, gw_ref[0], (((1,), (1,)), ((), ())),
                        preferred_element_type=jnp.float32)
    u = lax.dot_general(xb, uw_ref[0], (((1,), (1,)), ((), ())),
                        preferred_element_type=jnp.float32)
    h = ((g * jax.nn.sigmoid(g)) * u).astype(jnp.bfloat16)
    acc_ref[...] += lax.dot_general(h, dw_ref[0], (((1,), (1,)), ((), ())),
                                    preferred_element_type=jnp.float32)

    @pl.when(i == pl.num_programs(1) - 1)
    def _():
        ys_ref[...] = acc_ref[...]


def _pairsum_body(g_ref, p_ref, out_ref):
    g = g_ref[...]
    p = p_ref[...]
    out_ref[...] = (p[:, 0][:, None] * g[:, 0, :]
                    + p[:, 1][:, None] * g[:, 1, :])


def _make_sc_scatter(n_tok, n_out):
    nc = n_tok // (NW * CHUNK)
    mesh = plsc.VectorSubcoreMesh(core_axis_name="c", subcore_axis_name="s")

    @functools.partial(
        pl.kernel, mesh=mesh,
        out_type=jax.ShapeDtypeStruct((n_out, H), jnp.float32),
        scratch_types=[
            pltpu.VMEM((nc, CHUNK), jnp.int32),
            pltpu.VMEM((nc, CHUNK), jnp.int32),
            pltpu.VMEM((CHUNK, H), jnp.float32),
            pltpu.VMEM((CHUNK, H), jnp.float32),
            pltpu.SemaphoreType.DMA,
            pltpu.SemaphoreType.DMA,
            pltpu.SemaphoreType.DMA,
            pltpu.SemaphoreType.DMA,
        ],
    )
    def k(x_hbm, pe_hbm, po_hbm, out_hbm, pe_v, po_v, rows_v0, rows_v1,
          ls0, ls1, ss0, ss1):
        wid = lax.axis_index("s") * 2 + lax.axis_index("c")
        pltpu.sync_copy(pe_hbm.at[wid], pe_v)
        pltpu.sync_copy(po_hbm.at[wid], po_v)
        bufs = (rows_v0, rows_v1)
        lsems = (ls0, ls1)
        ssems = (ss0, ss1)
        base = wid * nc * CHUNK
        loads = [None] * nc
        stores = [None] * (2 * nc)
        loads[0] = pltpu.async_copy(
            x_hbm.at[pl.ds(base, CHUNK)], bufs[0], lsems[0])
        for c in range(nc):
            loads[c].wait()
            stores[2 * c] = pltpu.async_copy(
                bufs[c % 2], out_hbm.at[pe_v.at[c]], ssems[c % 2])
            stores[2 * c + 1] = pltpu.async_copy(
                bufs[c % 2], out_hbm.at[po_v.at[c]], ssems[c % 2])
            if c + 1 < nc:
                if c >= 1:
                    stores[2 * c - 2].wait()
                    stores[2 * c - 1].wait()
                loads[c + 1] = pltpu.async_copy(
                    x_hbm.at[pl.ds(base + (c + 1) * CHUNK, CHUNK)],
                    bufs[(c + 1) % 2], lsems[(c + 1) % 2])
        if nc >= 2:
            stores[2 * nc - 4].wait()
            stores[2 * nc - 3].wait()
        stores[2 * nc - 2].wait()
        stores[2 * nc - 1].wait()

    return k


def _make_sc_gather(n_rows):
    nc = n_rows // (NW * CHUNK)
    mesh = plsc.VectorSubcoreMesh(core_axis_name="c", subcore_axis_name="s")

    @functools.partial(
        pl.kernel, mesh=mesh,
        out_type=jax.ShapeDtypeStruct((n_rows, H), jnp.float32),
        scratch_types=[
            pltpu.VMEM((nc, CHUNK), jnp.int32),
            pltpu.VMEM((CHUNK, H), jnp.float32),
            pltpu.VMEM((CHUNK, H), jnp.float32),
            pltpu.SemaphoreType.DMA,
            pltpu.SemaphoreType.DMA,
            pltpu.SemaphoreType.DMA,
            pltpu.SemaphoreType.DMA,
        ],
    )
    def k(src_hbm, idx_hbm, out_hbm, idx_v, rows_v0, rows_v1, gs0, gs1,
          ss0, ss1):
        wid = lax.axis_index("s") * 2 + lax.axis_index("c")
        pltpu.sync_copy(idx_hbm.at[wid], idx_v)
        bufs = (rows_v0, rows_v1)
        gsems = (gs0, gs1)
        ssems = (ss0, ss1)
        gathers = [None] * nc
        stores = [None] * nc
        gathers[0] = pltpu.async_copy(src_hbm.at[idx_v.at[0]], bufs[0],
                                      gsems[0])
        for c in range(nc):
            gathers[c].wait()
            stores[c] = pltpu.async_copy(
                bufs[c % 2],
                out_hbm.at[pl.ds((wid * nc + c) * CHUNK, CHUNK)],
                ssems[c % 2])
            if c + 1 < nc:
                if c >= 1:
                    stores[c - 1].wait()
                gathers[c + 1] = pltpu.async_copy(
                    src_hbm.at[idx_v.at[c + 1]], bufs[(c + 1) % 2],
                    gsems[(c + 1) % 2])
        if nc >= 2:
            stores[nc - 2].wait()
        stores[nc - 1].wait()

    return k


@jax.jit
def _run(x, gate_w, gate_proj_w, up_proj_w, down_proj_w):
    T = x.shape[0]
    A = 2 * T
    A_BUF = A + E * BT
    G = A_BUF // BT
    NT = T // 512
    NI = I // IT

    logits, idx2, p2 = pl.pallas_call(
        _router_body,
        grid=(NT,),
        in_specs=[
            pl.BlockSpec((512, H), lambda t: (t, 0)),
            pl.BlockSpec((E, H), lambda t: (0, 0)),
        ],
        out_specs=[
            pl.BlockSpec((512, E), lambda t: (t, 0)),
            pl.BlockSpec((512, 2), lambda t: (t, 0)),
            pl.BlockSpec((512, 2), lambda t: (t, 0)),
        ],
        out_shape=[
            jax.ShapeDtypeStruct((T, E), jnp.float32),
            jax.ShapeDtypeStruct((T, 2), jnp.int32),
            jax.ShapeDtypeStruct((T, 2), jnp.float32),
        ],
    )(x, gate_w)

    a = idx2.reshape(-1)
    oh = (a[:, None] == jnp.arange(E, dtype=jnp.int32)).astype(jnp.int32)
    cum = jnp.cumsum(oh, axis=0)
    rank = jnp.take_along_axis(cum, a[:, None], axis=1)[:, 0] - 1
    counts = cum[-1]
    nb = (counts + BT - 1) // BT
    cnb = jnp.cumsum(nb)
    blk_start = cnb - nb
    pos = (blk_start[a] * BT + rank).astype(jnp.int32)
    block_expert = jnp.minimum(
        jnp.searchsorted(cnb, jnp.arange(G, dtype=jnp.int32), side="right"),
        E - 1).astype(jnp.int32)
    pos2 = pos.reshape(T, 2)

    xs = _make_sc_scatter(T, A_BUF)(
        x, pos2[:, 0].reshape(NW, -1, CHUNK),
        pos2[:, 1].reshape(NW, -1, CHUNK))

    ys = pl.pallas_call(
        _gmm_body,
        grid_spec=pltpu.PrefetchScalarGridSpec(
            num_scalar_prefetch=1,
            grid=(G, NI),
            in_specs=[
                pl.BlockSpec((BT, H), lambda g, i, be: (g, 0)),
                pl.BlockSpec((1, IT, H), lambda g, i, be: (be[g], i, 0)),
                pl.BlockSpec((1, IT, H), lambda g, i, be: (be[g], i, 0)),
                pl.BlockSpec((1, H, IT), lambda g, i, be: (be[g], 0, i)),
            ],
            out_specs=pl.BlockSpec((BT, H), lambda g, i, be: (g, 0)),
            scratch_shapes=[pltpu.VMEM((BT, H), jnp.float32)],
        ),
        out_shape=jax.ShapeDtypeStruct((A_BUF, H), jnp.float32),
        compiler_params=pltpu.CompilerParams(
            dimension_semantics=("arbitrary", "arbitrary")),
    )(block_expert, xs, gate_proj_w.astype(jnp.bfloat16),
      up_proj_w.astype(jnp.bfloat16), down_proj_w.astype(jnp.bfloat16))

    g3 = _make_sc_gather(A)(
        ys, pos.reshape(NW, -1, CHUNK)).reshape(T, 2, H)

    final = pl.pallas_call(
        _pairsum_body,
        grid=(NT,),
        in_specs=[
            pl.BlockSpec((512, 2, H), lambda t: (t, 0, 0)),
            pl.BlockSpec((512, 2), lambda t: (t, 0)),
        ],
        out_specs=pl.BlockSpec((512, H), lambda t: (t, 0)),
        out_shape=jax.ShapeDtypeStruct((T, H), jnp.float32),
    )(g3, p2)

    return final, logits


def kernel(hidden_state, gate_w, gate_proj_w, up_proj_w, down_proj_w):
    b, s, h = hidden_state.shape
    x = hidden_state.reshape(-1, h)
    final, logits = _run(x, gate_w, gate_proj_w, up_proj_w, down_proj_w)
    return final.reshape(b, s, h), logits


# speedup vs baseline: 1.6270x; 1.0385x over previous
import functools

import jax
import jax.numpy as jnp
from jax import lax
from jax.experimental import pallas as pl
from jax.experimental.pallas import tpu as pltpu
from jax.experimental.pallas import tpu_sc as plsc

E = 8
H = 1024
I = 2048
BT = 512
IT = 512
NW = 32
CHUNK = 32


def _router_body(x_ref, gw_ref, logits_ref, idx_ref, p_ref):
    x = x_ref[...]
    gw = gw_ref[...]
    logits = lax.dot_general(x, gw, (((1,), (1,)), ((), ())),
                             preferred_element_type=jnp.float32)
    logits_ref[...] = logits
    m = jnp.max(logits, axis=-1, keepdims=True)
    ex = jnp.exp(logits - m)
    p = ex / jnp.sum(ex, axis=-1, keepdims=True)
    lane = lax.broadcasted_iota(jnp.int32, p.shape, 1)
    m1 = jnp.max(p, axis=-1, keepdims=True)
    i1 = jnp.min(jnp.where(p == m1, lane, E), axis=-1, keepdims=True)
    p2m = jnp.where(lane == i1, -1.0, p)
    m2 = jnp.max(p2m, axis=-1, keepdims=True)
    i2 = jnp.min(jnp.where(p2m == m2, lane, E), axis=-1, keepdims=True)
    idx_ref[...] = jnp.concatenate([i1, i2], axis=1)
    p_ref[...] = jnp.concatenate([m1, m2], axis=1)


def _gmm_body(be_ref, xs_ref, gw_ref, uw_ref, dw_ref, ys_ref, acc_ref):
    gidx = pl.program_id(0)
    i = pl.program_id(1)
    nreal = be_ref[be_ref.shape[0] - 1]

    @pl.when(i == 0)
    def _():
        acc_ref[...] = jnp.zeros_like(acc_ref)

    @pl.when(gidx < nreal)
    def _():
        xb = xs_ref[...].astype(jnp.bfloat16)
        g = lax.dot_general(xb, gw_ref[0], (((1,), (1,)), ((), ())),
                            preferred_element_type=jnp.float32)
        u = lax.dot_general(xb, uw_ref[0], (((1,), (1,)), ((), ())),
                            preferred_element_type=jnp.float32)
        h = ((g * jax.nn.sigmoid(g)) * u).astype(jnp.bfloat16)
        acc_ref[...] += lax.dot_general(
            h, dw_ref[0], (((1,), (1,)), ((), ())),
            preferred_element_type=jnp.float32)

    @pl.when(i == pl.num_programs(1) - 1)
    def _():
        ys_ref[...] = acc_ref[...]


def _pairsum_body(g_ref, p_ref, out_ref):
    g = g_ref[...]
    p = p_ref[...]
    out_ref[...] = (p[:, 0][:, None] * g[:, 0, :]
                    + p[:, 1][:, None] * g[:, 1, :])


def _make_sc_scatter(n_tok, n_out):
    nc = n_tok // (NW * CHUNK)
    mesh = plsc.VectorSubcoreMesh(core_axis_name="c", subcore_axis_name="s")

    @functools.partial(
        pl.kernel, mesh=mesh,
        out_type=jax.ShapeDtypeStruct((n_out, H), jnp.float32),
        scratch_types=[
            pltpu.VMEM((nc, CHUNK), jnp.int32),
            pltpu.VMEM((nc, CHUNK), jnp.int32),
            pltpu.VMEM((CHUNK, H), jnp.float32),
            pltpu.VMEM((CHUNK, H), jnp.float32),
            pltpu.SemaphoreType.DMA,
            pltpu.SemaphoreType.DMA,
            pltpu.SemaphoreType.DMA,
            pltpu.SemaphoreType.DMA,
        ],
    )
    def k(x_hbm, pe_hbm, po_hbm, out_hbm, pe_v, po_v, rows_v0, rows_v1,
          ls0, ls1, ss0, ss1):
        wid = lax.axis_index("s") * 2 + lax.axis_index("c")
        pltpu.sync_copy(pe_hbm.at[wid], pe_v)
        pltpu.sync_copy(po_hbm.at[wid], po_v)
        bufs = (rows_v0, rows_v1)
        lsems = (ls0, ls1)
        ssems = (ss0, ss1)
        base = wid * nc * CHUNK
        loads = [None] * nc
        stores = [None] * (2 * nc)
        loads[0] = pltpu.async_copy(
            x_hbm.at[pl.ds(base, CHUNK)], bufs[0], lsems[0])
        for c in range(nc):
            loads[c].wait()
            stores[2 * c] = pltpu.async_copy(
                bufs[c % 2], out_hbm.at[pe_v.at[c]], ssems[c % 2])
            stores[2 * c + 1] = pltpu.async_copy(
                bufs[c % 2], out_hbm.at[po_v.at[c]], ssems[c % 2])
            if c + 1 < nc:
                if c >= 1:
                    stores[2 * c - 2].wait()
                    stores[2 * c - 1].wait()
                loads[c + 1] = pltpu.async_copy(
                    x_hbm.at[pl.ds(base + (c + 1) * CHUNK, CHUNK)],
                    bufs[(c + 1) % 2], lsems[(c + 1) % 2])
        if nc >= 2:
            stores[2 * nc - 4].wait()
            stores[2 * nc - 3].wait()
        stores[2 * nc - 2].wait()
        stores[2 * nc - 1].wait()

    return k


def _make_sc_gather(n_rows):
    nc = n_rows // (NW * CHUNK)
    mesh = plsc.VectorSubcoreMesh(core_axis_name="c", subcore_axis_name="s")

    @functools.partial(
        pl.kernel, mesh=mesh,
        out_type=jax.ShapeDtypeStruct((n_rows, H), jnp.float32),
        scratch_types=[
            pltpu.VMEM((nc, CHUNK), jnp.int32),
            pltpu.VMEM((CHUNK, H), jnp.float32),
            pltpu.VMEM((CHUNK, H), jnp.float32),
            pltpu.SemaphoreType.DMA,
            pltpu.SemaphoreType.DMA,
            pltpu.SemaphoreType.DMA,
            pltpu.SemaphoreType.DMA,
        ],
    )
    def k(src_hbm, idx_hbm, out_hbm, idx_v, rows_v0, rows_v1, gs0, gs1,
          ss0, ss1):
        wid = lax.axis_index("s") * 2 + lax.axis_index("c")
        pltpu.sync_copy(idx_hbm.at[wid], idx_v)
        bufs = (rows_v0, rows_v1)
        gsems = (gs0, gs1)
        ssems = (ss0, ss1)
        gathers = [None] * nc
        stores = [None] * nc
        gathers[0] = pltpu.async_copy(src_hbm.at[idx_v.at[0]], bufs[0],
                                      gsems[0])
        for c in range(nc):
            gathers[c].wait()
            stores[c] = pltpu.async_copy(
                bufs[c % 2],
                out_hbm.at[pl.ds((wid * nc + c) * CHUNK, CHUNK)],
                ssems[c % 2])
            if c + 1 < nc:
                if c >= 1:
                    stores[c - 1].wait()
                gathers[c + 1] = pltpu.async_copy(
                    src_hbm.at[idx_v.at[c + 1]], bufs[(c + 1) % 2],
                    gsems[(c + 1) % 2])
        if nc >= 2:
            stores[nc - 2].wait()
        stores[nc - 1].wait()

    return k


@jax.jit
def _run(x, gate_w, gate_proj_w, up_proj_w, down_proj_w):
    T = x.shape[0]
    A = 2 * T
    A_BUF = A + E * BT
    G = A_BUF // BT
    NT = T // 512
    NI = I // IT

    logits, idx2, p2 = pl.pallas_call(
        _router_body,
        grid=(NT,),
        in_specs=[
            pl.BlockSpec((512, H), lambda t: (t, 0)),
            pl.BlockSpec((E, H), lambda t: (0, 0)),
        ],
        out_specs=[
            pl.BlockSpec((512, E), lambda t: (t, 0)),
            pl.BlockSpec((512, 2), lambda t: (t, 0)),
            pl.BlockSpec((512, 2), lambda t: (t, 0)),
        ],
        out_shape=[
            jax.ShapeDtypeStruct((T, E), jnp.float32),
            jax.ShapeDtypeStruct((T, 2), jnp.int32),
            jax.ShapeDtypeStruct((T, 2), jnp.float32),
        ],
    )(x, gate_w)

    a = idx2.reshape(-1)
    oh = (a[:, None] == jnp.arange(E, dtype=jnp.int32)).astype(jnp.int32)
    cum = jnp.cumsum(oh, axis=0)
    rank = jnp.take_along_axis(cum, a[:, None], axis=1)[:, 0] - 1
    counts = cum[-1]
    nb = (counts + BT - 1) // BT
    cnb = jnp.cumsum(nb)
    blk_start = cnb - nb
    pos = (blk_start[a] * BT + rank).astype(jnp.int32)
    block_expert = jnp.minimum(
        jnp.searchsorted(cnb, jnp.arange(G, dtype=jnp.int32), side="right"),
        E - 1).astype(jnp.int32)
    block_expert = jnp.concatenate(
        [block_expert, cnb[-1:].astype(jnp.int32)])
    pos2 = pos.reshape(T, 2)

    xs = _make_sc_scatter(T, A_BUF)(
        x, pos2[:, 0].reshape(NW, -1, CHUNK),
        pos2[:, 1].reshape(NW, -1, CHUNK))

    ys = pl.pallas_call(
        _gmm_body,
        grid_spec=pltpu.PrefetchScalarGridSpec(
            num_scalar_prefetch=1,
            grid=(G, NI),
            in_specs=[
                pl.BlockSpec((BT, H), lambda g, i, be: (g, 0)),
                pl.BlockSpec((1, IT, H), lambda g, i, be: (be[g], i, 0)),
                pl.BlockSpec((1, IT, H), lambda g, i, be: (be[g], i, 0)),
                pl.BlockSpec((1, H, IT), lambda g, i, be: (be[g], 0, i)),
            ],
            out_specs=pl.BlockSpec((BT, H), lambda g, i, be: (g, 0)),
            scratch_shapes=[pltpu.VMEM((BT, H), jnp.float32)],
        ),
        out_shape=jax.ShapeDtypeStruct((A_BUF, H), jnp.float32),
        compiler_params=pltpu.CompilerParams(
            dimension_semantics=("arbitrary", "arbitrary")),
    )(block_expert, xs, gate_proj_w.astype(jnp.bfloat16),
      up_proj_w.astype(jnp.bfloat16), down_proj_w.astype(jnp.bfloat16))

    g3 = _make_sc_gather(A)(
        ys, pos.reshape(NW, -1, CHUNK)).reshape(T, 2, H)

    final = pl.pallas_call(
        _pairsum_body,
        grid=(NT,),
        in_specs=[
            pl.BlockSpec((512, 2, H), lambda t: (t, 0, 0)),
            pl.BlockSpec((512, 2), lambda t: (t, 0)),
        ],
        out_specs=pl.BlockSpec((512, H), lambda t: (t, 0)),
        out_shape=jax.ShapeDtypeStruct((T, H), jnp.float32),
    )(g3, p2)

    return final, logits


def kernel(hidden_state, gate_w, gate_proj_w, up_proj_w, down_proj_w):
    b, s, h = hidden_state.shape
    x = hidden_state.reshape(-1, h)
    final, logits = _run(x, gate_w, gate_proj_w, up_proj_w, down_proj_w)
    return final.reshape(b, s, h), logits
